# Initial kernel scaffold; baseline (speedup 1.0000x reference)
import jax
import jax.numpy as jnp
from jax.experimental import pallas as pl


def kernel(user_id, gender_id, job_id, city_id, age_bucket, user_label_list, item_id, category_id, item_city_id, item_label_list, user_id_emb, gender_emb, job_emb, city_emb, age_emb, item_id_emb, category_emb, label_emb, W_u1, b_u1, W_u2, b_u2, W_i1, b_i1, W_i2, b_i2):
    raise NotImplementedError("write your pallas kernel here")


# speedup vs baseline: 4.6434x; 4.6434x over previous
import jax
import jax.numpy as jnp
from jax import lax
from jax.experimental import pallas as pl
from jax.experimental.pallas import tpu as pltpu
from jax.experimental.pallas import tpu_sc as plsc

B = 16384
L = 20
NC, NS = 2, 16
NW = NC * NS
BW = B // NW

CT = 64


def _sc_wide_body(uid, iid, user_id_emb, item_id_emb, o_uid, o_iid,
                  uid_v, iid_v, r_uid, r_iid, sem_i, sem_d):
    wid = lax.axis_index("s") * NC + lax.axis_index("c")
    lanes = lax.iota(jnp.int32, 16)

    def chunk_body(ch, carry):
        base = pl.multiple_of(wid * BW + ch * CT, CT)
        pltpu.async_copy(uid.at[pl.ds(base, CT)], uid_v, sem_i).wait()
        pltpu.async_copy(iid.at[pl.ds(base, CT)], iid_v, sem_i).wait()

        def row_body(s, carry2):
            grp = s // 16
            lane = s % 16
            msk = lanes == lane
            r_u = jnp.sum(jnp.where(msk, uid_v[pl.ds(grp * 16, 16)], 0))
            r_i = jnp.sum(jnp.where(msk, iid_v[pl.ds(grp * 16, 16)], 0))
            pltpu.async_copy(user_id_emb.at[r_u], r_uid.at[s], sem_d)
            pltpu.async_copy(item_id_emb.at[r_i], r_iid.at[s], sem_d)
            return carry2

        lax.fori_loop(0, CT, row_body, 0, unroll=False)

        def drain(s, carry2):
            pltpu.make_async_copy(user_id_emb.at[0], r_uid.at[s], sem_d).wait()
            pltpu.make_async_copy(item_id_emb.at[0], r_iid.at[s], sem_d).wait()
            return carry2

        lax.fori_loop(0, CT, drain, 0, unroll=False)
        pltpu.async_copy(r_uid, o_uid.at[pl.ds(base, CT)], sem_i).wait()
        pltpu.async_copy(r_iid, o_iid.at[pl.ds(base, CT)], sem_i).wait()
        return carry

    lax.fori_loop(0, BW // CT, chunk_body, 0, unroll=False)


def _make_sc_wide():
    mesh = plsc.VectorSubcoreMesh(core_axis_name="c", subcore_axis_name="s",
                                  num_cores=NC, num_subcores=NS)
    f32, i32 = jnp.float32, jnp.int32
    out_type = (
        jax.ShapeDtypeStruct((B, 64), f32),
        jax.ShapeDtypeStruct((B, 64), f32),
    )
    scratch = [
        pltpu.VMEM((CT,), i32), pltpu.VMEM((CT,), i32),
        pltpu.VMEM((CT, 64), f32), pltpu.VMEM((CT, 64), f32),
        pltpu.SemaphoreType.DMA, pltpu.SemaphoreType.DMA,
    ]
    return pl.kernel(_sc_wide_body, out_type=out_type, mesh=mesh,
                     scratch_types=scratch,
                     compiler_params=pltpu.CompilerParams(
                         needs_layout_passes=False))


_sc_wide = _make_sc_wide()

C = 32
NCHUNK = BW // C
NPAIR = NCHUNK // 2
CL = C * L
IDX_STEP = 128
NSM = 6


def _sc_small_body(ids6, ulbl, ilbl,
                   gender_emb, job_emb, city_emb, age_emb, category_emb,
                   label_emb,
                   o_g, o_j, o_c, o_a, o_ic, o_cat, o_ul, o_il,
                   ids_v0, ids_v1, ulbl_v0, ulbl_v1, ilbl_v0, ilbl_v1,
                   r_g0, r_g1, r_j0, r_j1, r_c0, r_c1, r_a0, r_a1,
                   r_ic0, r_ic1, r_cat0, r_cat1,
                   r_ul0, r_ul1, r_il0, r_il1, m_ul0, m_ul1, m_il0, m_il1,
                   sem_i, sem_g0, sem_g1, sem_w0, sem_w1):
    wid = lax.axis_index("s") * NC + lax.axis_index("c")
    outs = (o_g, o_j, o_c, o_a, o_ic, o_cat, o_ul, o_il)
    ids_v = (ids_v0, ids_v1)
    ulbl_v, ilbl_v = (ulbl_v0, ulbl_v1), (ilbl_v0, ilbl_v1)
    r_g, r_j, r_c, r_a = (r_g0, r_g1), (r_j0, r_j1), (r_c0, r_c1), (r_a0, r_a1)
    r_ic, r_cat = (r_ic0, r_ic1), (r_cat0, r_cat1)
    r_ul, r_il = (r_ul0, r_ul1), (r_il0, r_il1)
    m_ul, m_il = (m_ul0, m_ul1), (m_il0, m_il1)
    sem_g = (sem_g0, sem_g1)
    sem_w = (sem_w0, sem_w1)

    def stage_idx(bf, ch):
        base = pl.multiple_of(wid * BW + ch * C, C)
        lbase = pl.multiple_of(base * L, C * L)
        cps = [
            pltpu.async_copy(ids6.at[:, pl.ds(base, C)], ids_v[bf], sem_i),
            pltpu.async_copy(ulbl.at[pl.ds(lbase, CL)], ulbl_v[bf], sem_i),
            pltpu.async_copy(ilbl.at[pl.ds(lbase, CL)], ilbl_v[bf], sem_i),
        ]
        for cp in cps:
            cp.wait()

    def fire_gathers(bf):
        iv = ids_v[bf]
        sem = sem_g[bf]
        pltpu.async_copy(gender_emb.at[iv.at[0]], r_g[bf], sem)
        pltpu.async_copy(job_emb.at[iv.at[1]], r_j[bf], sem)
        pltpu.async_copy(city_emb.at[iv.at[2]], r_c[bf], sem)
        pltpu.async_copy(age_emb.at[iv.at[3]], r_a[bf], sem)
        pltpu.async_copy(city_emb.at[iv.at[4]], r_ic[bf], sem)
        pltpu.async_copy(category_emb.at[iv.at[5]], r_cat[bf], sem)
        for k in range(CL // IDX_STEP):
            sl = pl.ds(k * IDX_STEP, IDX_STEP)
            pltpu.async_copy(label_emb.at[ulbl_v[bf].at[sl]],
                             r_ul[bf].at[sl], sem)
            pltpu.async_copy(label_emb.at[ilbl_v[bf].at[sl]],
                             r_il[bf].at[sl], sem)

    def drain_gathers(bf):
        sem = sem_g[bf]
        pltpu.make_async_copy(city_emb.at[pl.ds(0, C)], r_g[bf], sem).wait()
        pltpu.make_async_copy(city_emb.at[pl.ds(0, C)], r_j[bf], sem).wait()
        pltpu.make_async_copy(city_emb.at[pl.ds(0, C)], r_c[bf], sem).wait()
        pltpu.make_async_copy(city_emb.at[pl.ds(0, C)], r_a[bf], sem).wait()
        pltpu.make_async_copy(city_emb.at[pl.ds(0, C)], r_ic[bf], sem).wait()
        pltpu.make_async_copy(category_emb.at[pl.ds(0, C)], r_cat[bf],
                              sem).wait()
        for k in range(CL // IDX_STEP):
            sl = pl.ds(k * IDX_STEP, IDX_STEP)
            pltpu.make_async_copy(label_emb.at[pl.ds(0, IDX_STEP)],
                                  r_ul[bf].at[sl], sem).wait()
            pltpu.make_async_copy(label_emb.at[pl.ds(0, IDX_STEP)],
                                  r_il[bf].at[sl], sem).wait()

    def pool(bf):
        inv = jnp.float32(1.0 / L)
        rul, ril = r_ul[bf], r_il[bf]
        mul_, mil_ = m_ul[bf], m_il[bf]

        def pbody(s, carry):
            for rows, mean in ((rul, mul_), (ril, mil_)):
                for h in range(2):
                    acc = rows[s * L, pl.ds(h * 16, 16)]
                    for j in range(1, L):
                        acc = acc + rows[s * L + j, pl.ds(h * 16, 16)]
                    mean[s, pl.ds(h * 16, 16)] = acc * inv
            return carry

        lax.fori_loop(0, C, pbody, 0, unroll=False)

    def pieces(bf):
        return (r_g[bf], r_j[bf], r_c[bf], r_a[bf], r_ic[bf], r_cat[bf],
                m_ul[bf], m_il[bf])

    def fire_writes(bf, ch):
        base = pl.multiple_of(wid * BW + ch * C, C)
        for src, dst in zip(pieces(bf), outs):
            pltpu.async_copy(src, dst.at[pl.ds(base, C)], sem_w[bf])

    def drain_writes(bf):
        for src, dst in zip(pieces(bf), outs):
            pltpu.make_async_copy(src, dst.at[pl.ds(0, C)], sem_w[bf]).wait()

    stage_idx(0, 0)
    fire_gathers(0)

    def pair_body(t, carry):
        c0 = t * 2
        stage_idx(1, c0 + 1)

        @pl.when(t > 0)
        def _():
            drain_writes(1)
        fire_gathers(1)
        drain_gathers(0)
        pool(0)
        fire_writes(0, c0)

        @pl.when(t + 1 < NPAIR)
        def _():
            stage_idx(0, c0 + 2)
            drain_writes(0)
            fire_gathers(0)
        drain_gathers(1)
        pool(1)
        fire_writes(1, c0 + 1)
        return carry

    lax.fori_loop(0, NPAIR, pair_body, 0, unroll=False)
    drain_writes(0)
    drain_writes(1)


def _make_sc_small():
    mesh = plsc.VectorSubcoreMesh(core_axis_name="c", subcore_axis_name="s",
                                  num_cores=NC, num_subcores=NS)
    f32, i32 = jnp.float32, jnp.int32
    out_type = (
        jax.ShapeDtypeStruct((B, 16), f32),
        jax.ShapeDtypeStruct((B, 16), f32),
        jax.ShapeDtypeStruct((B, 16), f32),
        jax.ShapeDtypeStruct((B, 16), f32),
        jax.ShapeDtypeStruct((B, 16), f32),
        jax.ShapeDtypeStruct((B, 32), f32),
        jax.ShapeDtypeStruct((B, 32), f32),
        jax.ShapeDtypeStruct((B, 32), f32),
    )

    def dbl(shape, dt):
        return [pltpu.VMEM(shape, dt), pltpu.VMEM(shape, dt)]

    scratch = (
        dbl((NSM, C), i32) + dbl((CL,), i32) + dbl((CL,), i32) +
        dbl((C, 16), f32) + dbl((C, 16), f32) + dbl((C, 16), f32) +
        dbl((C, 16), f32) + dbl((C, 16), f32) + dbl((C, 32), f32) +
        dbl((CL, 32), f32) + dbl((CL, 32), f32) +
        dbl((C, 32), f32) + dbl((C, 32), f32) +
        [pltpu.SemaphoreType.DMA] * 5
    )
    return pl.kernel(_sc_small_body, out_type=out_type, mesh=mesh,
                     scratch_types=scratch,
                     compiler_params=pltpu.CompilerParams(
                         use_tc_tiling_on_sc=False))


_sc_small = _make_sc_small()

BM = 2048


def _tc_body(uidr, gr, jr, cr, ar, ulr, iidr, catr, icr, ilr,
             wu1, bu1, wu2, bu2, wi1, bi1, wi2, bi2, out):
    uf = jnp.concatenate(
        [uidr[...], gr[...], jr[...], cr[...], ar[...], ulr[...]], axis=1)
    hu = jnp.maximum(
        jnp.dot(uf, wu1[...], preferred_element_type=jnp.float32) + bu1[...],
        0.0)
    uv = jnp.dot(hu, wu2[...], preferred_element_type=jnp.float32) + bu2[...]
    itf = jnp.concatenate([iidr[...], catr[...], icr[...], ilr[...]], axis=1)
    hi = jnp.dot(itf, wi1[...], preferred_element_type=jnp.float32) + bi1[...]
    iv = jnp.dot(hi, wi2[...], preferred_element_type=jnp.float32) + bi2[...]
    out[...] = jax.nn.sigmoid(jnp.sum(uv * iv, axis=1))


def _tc_towers(pieces, wu1, bu1, wu2, bu2, wi1, bi1, wi2, bi2):
    grid = (B // BM,)
    row = lambda w: pl.BlockSpec((BM, w), lambda i: (i, 0))
    full = lambda a: pl.BlockSpec(a.shape, lambda i: (0,) * a.ndim)
    piece_specs = [row(p.shape[1]) for p in pieces]
    w_specs = [full(a) for a in (wu1, bu1, wu2, bu2, wi1, bi1, wi2, bi2)]
    return pl.pallas_call(
        _tc_body,
        grid=grid,
        in_specs=piece_specs + w_specs,
        out_specs=pl.BlockSpec((BM,), lambda i: (i,)),
        out_shape=jax.ShapeDtypeStruct((B,), jnp.float32),
    )(*pieces, wu1, bu1, wu2, bu2, wi1, bi1, wi2, bi2)


def kernel(user_id, gender_id, job_id, city_id, age_bucket, user_label_list,
           item_id, category_id, item_city_id, item_label_list,
           user_id_emb, gender_emb, job_emb, city_emb, age_emb,
           item_id_emb, category_emb, label_emb,
           W_u1, b_u1, W_u2, b_u2, W_i1, b_i1, W_i2, b_i2):
    i32 = jnp.int32
    u_id, i_id = _sc_wide(user_id.astype(i32), item_id.astype(i32),
                          user_id_emb, item_id_emb)
    ids6 = jnp.stack([
        gender_id.astype(i32), job_id.astype(i32), city_id.astype(i32),
        age_bucket.astype(i32), item_city_id.astype(i32),
        category_id.astype(i32)])
    g, j, c, a, ic, cat, ul, il = _sc_small(
        ids6,
        user_label_list.astype(i32).reshape(B * L),
        item_label_list.astype(i32).reshape(B * L),
        gender_emb, job_emb, city_emb, age_emb, category_emb, label_emb)
    pieces = (u_id, g, j, c, a, ul, i_id, cat, ic, il)
    return _tc_towers(pieces, W_u1, b_u1.reshape(1, 256), W_u2,
                      b_u2.reshape(1, 128), W_i1, b_i1.reshape(1, 256),
                      W_i2, b_i2.reshape(1, 128))
